# baseline (speedup 1.0000x reference)
import functools

import jax
import jax.numpy as jnp
from jax import lax
from jax.experimental import pallas as pl
from jax.experimental.pallas import tpu as pltpu

N = 10000
E = 320000
G = 256
DF = 128
DE = 16
EO = 256
NO = 256
EX = 32
HID = 256
NEXP = 8

BE = 512
BN = 400


def _mm_body(x_ref, w_ref, o_ref):
    o_ref[...] = jnp.dot(x_ref[...], w_ref[...],
                         preferred_element_type=jnp.float32)


def _mm(x, w, block_rows):
    m, k = x.shape
    n = w.shape[1]
    grid = m // block_rows
    return pl.pallas_call(
        _mm_body,
        grid=(grid,),
        in_specs=[
            pl.BlockSpec((block_rows, k), lambda i: (i, 0)),
            pl.BlockSpec((k, n), lambda i: (0, 0)),
        ],
        out_specs=pl.BlockSpec((block_rows, n), lambda i: (i, 0)),
        out_shape=jax.ShapeDtypeStruct((m, n), jnp.float32),
    )(x, w)


def _h0_body(xws_ref, ef_ref, web_ref, o_ref):
    o_ref[...] = jnp.maximum(
        xws_ref[...] + jnp.dot(ef_ref[...], web_ref[...],
                               preferred_element_type=jnp.float32), 0.0)


def _h0(xw_src, ef, web):
    grid = E // BE
    return pl.pallas_call(
        _h0_body,
        grid=(grid,),
        in_specs=[
            pl.BlockSpec((BE, EO), lambda i: (i, 0)),
            pl.BlockSpec((BE, DE), lambda i: (i, 0)),
            pl.BlockSpec((DE, EO), lambda i: (0, 0)),
        ],
        out_specs=pl.BlockSpec((BE, EO), lambda i: (i, 0)),
        out_shape=jax.ShapeDtypeStruct((E, EO), jnp.float32),
    )(xw_src, ef, web)


def _round_body(g1_ref, g2_ref, h0_ref, wu_ref, o_ref):
    m = g1_ref[...] - g2_ref[...]
    o_ref[...] = jnp.maximum(
        jnp.dot(m, wu_ref[...], preferred_element_type=jnp.float32)
        + h0_ref[...], 0.0)


def _round(g1, g2, h0, wu):
    grid = E // BE
    return pl.pallas_call(
        _round_body,
        grid=(grid,),
        in_specs=[
            pl.BlockSpec((BE, EO), lambda i: (i, 0)),
            pl.BlockSpec((BE, EO), lambda i: (i, 0)),
            pl.BlockSpec((BE, EO), lambda i: (i, 0)),
            pl.BlockSpec((EO, EO), lambda i: (0, 0)),
        ],
        out_specs=pl.BlockSpec((BE, EO), lambda i: (i, 0)),
        out_shape=jax.ShapeDtypeStruct((E, EO), jnp.float32),
    )(g1, g2, h0, wu)


def _node_body(x_ref, nm_ref, wa_ref, wb_ref, bn_ref, o_ref):
    acc = jnp.dot(x_ref[...], wa_ref[...], preferred_element_type=jnp.float32)
    acc += jnp.dot(nm_ref[...], wb_ref[...], preferred_element_type=jnp.float32)
    o_ref[...] = jnp.maximum(acc + bn_ref[...], 0.0)


def _node(x, node_m, wa, wb, bn):
    grid = N // BN
    return pl.pallas_call(
        _node_body,
        grid=(grid,),
        in_specs=[
            pl.BlockSpec((BN, DF), lambda i: (i, 0)),
            pl.BlockSpec((BN, EO), lambda i: (i, 0)),
            pl.BlockSpec((DF, NO), lambda i: (0, 0)),
            pl.BlockSpec((EO, NO), lambda i: (0, 0)),
            pl.BlockSpec((1, NO), lambda i: (0, 0)),
        ],
        out_specs=pl.BlockSpec((BN, NO), lambda i: (i, 0)),
        out_shape=jax.ShapeDtypeStruct((N, NO), jnp.float32),
    )(x, node_m, wa, wb, bn)


def _moe_body(c_ref, we1_ref, be1_ref, we2_ref, be2_ref, we3_ref, be3_ref,
              wg1_ref, bg1_ref, wg2_ref, bg2_ref, wg3_ref, bg3_ref, o_ref):
    c = c_ref[...]
    g = jnp.maximum(jnp.dot(c, wg1_ref[...],
                            preferred_element_type=jnp.float32)
                    + bg1_ref[...], 0.0)
    g = jnp.maximum(jnp.dot(g, wg2_ref[...],
                            preferred_element_type=jnp.float32)
                    + bg2_ref[...], 0.0)
    logits = jnp.dot(g, wg3_ref[...],
                     preferred_element_type=jnp.float32) + bg3_ref[...]
    gate = jax.nn.softmax(logits, axis=1)
    acc = jnp.zeros((G, 1), jnp.float32)
    for e in range(NEXP):
        t = jnp.maximum(jnp.dot(c, we1_ref[e],
                                preferred_element_type=jnp.float32)
                        + be1_ref[e][None, :], 0.0)
        t = jnp.maximum(jnp.dot(t, we2_ref[e],
                                preferred_element_type=jnp.float32)
                        + be2_ref[e][None, :], 0.0)
        t = jnp.dot(t, we3_ref[e],
                    preferred_element_type=jnp.float32) + be3_ref[e][None, :]
        acc += t * gate[:, e:e + 1]
    o_ref[...] = acc


def _moe(c, p):
    in_feat = c.shape[1]
    full = lambda *s: pl.BlockSpec(s, lambda: tuple(0 for _ in s))
    return pl.pallas_call(
        _moe_body,
        in_specs=[
            full(G, in_feat),
            full(NEXP, in_feat, HID), full(NEXP, HID),
            full(NEXP, HID, HID), full(NEXP, HID),
            full(NEXP, HID, 1), full(NEXP, 1),
            full(in_feat, HID), full(1, HID),
            full(HID, HID), full(1, HID),
            full(HID, NEXP), full(1, NEXP),
        ],
        out_specs=full(G, 1),
        out_shape=jax.ShapeDtypeStruct((G, 1), jnp.float32),
    )(c, p['We1'], p['be1'], p['We2'], p['be2'], p['We3'], p['be3'],
      p['Wg1'], p['bg1'].reshape(1, HID), p['Wg2'], p['bg2'].reshape(1, HID),
      p['Wg3'], p['bg3'].reshape(1, NEXP))


def _mpnn(x, ef, src, dst, rev, we, wu, wn, bn, rounds):
    we_top, we_bot = we[:DF], we[DF:]
    xw = _mm(x, we_top, BN)
    h0 = _h0(xw[src], ef, we_bot)
    h = h0
    for _ in range(rounds):
        sum0 = jax.ops.segment_sum(h, dst, num_segments=N)
        h = _round(sum0[src], h[rev], h0, wu)
    node_m = jax.ops.segment_sum(h, dst, num_segments=N)
    return _node(x, node_m, wn[:DF], wn[DF:], bn.reshape(1, NO))


def kernel(x_su, ef_su, src_su, dst_su, rev_su, gid_su,
           x_sv, ef_sv, src_sv, dst_sv, rev_sv, gid_sv,
           extra, params):
    p = params
    h_su = _mpnn(x_su, ef_su, src_su, dst_su, rev_su,
                 p['We_su'], p['Wu_su'], p['Wn_su'], p['bn_su'], 3)
    h_sv = _mpnn(x_sv, ef_sv, src_sv, dst_sv, rev_sv,
                 p['We_sv'], p['Wu_sv'], p['Wn_sv'], p['bn_sv'], 3)
    solute = jax.ops.segment_sum(h_su, gid_su, num_segments=G)
    solvent = jax.ops.segment_sum(h_sv, gid_sv, num_segments=G)
    combined = jnp.concatenate([solute, solvent, extra], axis=-1)
    return _moe(combined, p)
